# baseline (speedup 1.0000x reference)
import jax
import jax.numpy as jnp
from jax.experimental import pallas as pl

N = 256
F_IN = 128
F_OUT = 256
BI = 8


def _fused_body(x1_ref, x2_ref, w_ref, b_ref, out_ref):
    w = w_ref[...]
    bb = b_ref[0]
    i = pl.program_id(0)
    e1b = jax.lax.dot_general(
        x1_ref[...], w, (((2,), (0,)), ((), ())),
        preferred_element_type=jnp.float32) + bb
    x2blk = x2_ref[:, pl.ds(i * BI, BI), :]
    e2b = jax.lax.dot_general(
        x2blk, w, (((2,), (0,)), ((), ())),
        preferred_element_type=jnp.float32) + bb
    out_ref[...] = jax.lax.dot_general(
        e1b, e2b, (((2,), (0,)), ((0,), (1,))),
        preferred_element_type=jnp.float32)


def kernel(x, W, b):
    x1 = x[0, 0]
    x2 = x[0, 1]
    b2 = b.reshape(1, F_OUT)
    out = pl.pallas_call(
        _fused_body,
        grid=(N // BI,),
        in_specs=[
            pl.BlockSpec((BI, N, F_IN), lambda i: (i, 0, 0)),
            pl.BlockSpec((N, N, F_IN), lambda i: (0, 0, 0)),
            pl.BlockSpec((F_IN, F_OUT), lambda i: (0, 0)),
            pl.BlockSpec((1, F_OUT), lambda i: (0, 0)),
        ],
        out_specs=pl.BlockSpec((BI, N, F_OUT), lambda i: (i, 0, 0)),
        out_shape=jax.ShapeDtypeStruct((N, N, F_OUT), jnp.float32),
    )(x1, x2, W, b2)
    return out[None]


# speedup vs baseline: 1.0003x; 1.0003x over previous
import jax
import jax.numpy as jnp
from jax.experimental import pallas as pl

N = 256
F_IN = 128
F_OUT = 256
BI = 8


def _fused_body(x1_ref, x2_ref, w_ref, b_ref, out_ref):
    w = w_ref[...].astype(jnp.bfloat16)
    bb = b_ref[0]
    i = pl.program_id(0)
    e1b = jax.lax.dot_general(
        x1_ref[...].astype(jnp.bfloat16), w, (((2,), (0,)), ((), ())),
        preferred_element_type=jnp.float32) + bb
    x2blk = x2_ref[:, pl.ds(i * BI, BI), :].astype(jnp.bfloat16)
    e2b = jax.lax.dot_general(
        x2blk, w, (((2,), (0,)), ((), ())),
        preferred_element_type=jnp.float32) + bb
    out_ref[...] = jax.lax.dot_general(
        e1b.astype(jnp.bfloat16), e2b.astype(jnp.bfloat16),
        (((2,), (0,)), ((0,), (1,))),
        preferred_element_type=jnp.float32)


def kernel(x, W, b):
    x1 = x[0, 0]
    x2 = x[0, 1]
    b2 = b.reshape(1, F_OUT)
    out = pl.pallas_call(
        _fused_body,
        grid=(N // BI,),
        in_specs=[
            pl.BlockSpec((BI, N, F_IN), lambda i: (i, 0, 0)),
            pl.BlockSpec((N, N, F_IN), lambda i: (0, 0, 0)),
            pl.BlockSpec((F_IN, F_OUT), lambda i: (0, 0)),
            pl.BlockSpec((1, F_OUT), lambda i: (0, 0)),
        ],
        out_specs=pl.BlockSpec((BI, N, F_OUT), lambda i: (i, 0, 0)),
        out_shape=jax.ShapeDtypeStruct((N, N, F_OUT), jnp.float32),
    )(x1, x2, W, b2)
    return out[None]


# speedup vs baseline: 1.1069x; 1.1066x over previous
import jax
import jax.numpy as jnp
from jax.experimental import pallas as pl

N = 256
F_IN = 128
F_OUT = 256
BI = 16


def _fused_body(x1_ref, x2_ref, w_ref, b_ref, out_ref):
    w = w_ref[...].astype(jnp.bfloat16)
    bb = b_ref[0]
    i = pl.program_id(0)
    e1b = jax.lax.dot_general(
        x1_ref[...].astype(jnp.bfloat16), w, (((2,), (0,)), ((), ())),
        preferred_element_type=jnp.float32) + bb
    x2blk = x2_ref[:, pl.ds(i * BI, BI), :].astype(jnp.bfloat16)
    e2b = jax.lax.dot_general(
        x2blk, w, (((2,), (0,)), ((), ())),
        preferred_element_type=jnp.float32) + bb
    out_ref[...] = jax.lax.dot_general(
        e1b.astype(jnp.bfloat16), e2b.astype(jnp.bfloat16),
        (((2,), (0,)), ((0,), (1,))),
        preferred_element_type=jnp.float32)


def kernel(x, W, b):
    x1 = x[0, 0]
    x2 = x[0, 1]
    b2 = b.reshape(1, F_OUT)
    out = pl.pallas_call(
        _fused_body,
        grid=(N // BI,),
        in_specs=[
            pl.BlockSpec((BI, N, F_IN), lambda i: (i, 0, 0)),
            pl.BlockSpec((N, N, F_IN), lambda i: (0, 0, 0)),
            pl.BlockSpec((F_IN, F_OUT), lambda i: (0, 0)),
            pl.BlockSpec((1, F_OUT), lambda i: (0, 0)),
        ],
        out_specs=pl.BlockSpec((BI, N, F_OUT), lambda i: (i, 0, 0)),
        out_shape=jax.ShapeDtypeStruct((N, N, F_OUT), jnp.float32),
    )(x1, x2, W, b2)
    return out[None]


# speedup vs baseline: 1.1086x; 1.0016x over previous
import jax
import jax.numpy as jnp
from jax.experimental import pallas as pl
from jax.experimental.pallas import tpu as pltpu

N = 256
F_IN = 128
F_OUT = 256
BI = 16


def _fused_body(x1_ref, x2_ref, w_ref, b_ref, out_ref):
    w = w_ref[...].astype(jnp.bfloat16)
    bb = b_ref[0]
    i = pl.program_id(0)
    e1b = jax.lax.dot_general(
        x1_ref[...].astype(jnp.bfloat16), w, (((2,), (0,)), ((), ())),
        preferred_element_type=jnp.float32) + bb
    x2blk = x2_ref[:, pl.ds(i * BI, BI), :].astype(jnp.bfloat16)
    e2b = jax.lax.dot_general(
        x2blk, w, (((2,), (0,)), ((), ())),
        preferred_element_type=jnp.float32) + bb
    out_ref[...] = jax.lax.dot_general(
        e1b.astype(jnp.bfloat16), e2b.astype(jnp.bfloat16),
        (((2,), (0,)), ((0,), (1,))),
        preferred_element_type=jnp.float32)


def kernel(x, W, b):
    x1 = x[0, 0]
    x2 = x[0, 1]
    b2 = b.reshape(1, F_OUT)
    out = pl.pallas_call(
        _fused_body,
        grid=(N // BI,),
        in_specs=[
            pl.BlockSpec((BI, N, F_IN), lambda i: (i, 0, 0)),
            pl.BlockSpec((N, N, F_IN), lambda i: (0, 0, 0)),
            pl.BlockSpec((F_IN, F_OUT), lambda i: (0, 0)),
            pl.BlockSpec((1, F_OUT), lambda i: (0, 0)),
        ],
        out_specs=pl.BlockSpec((BI, N, F_OUT), lambda i: (i, 0, 0)),
        out_shape=jax.ShapeDtypeStruct((N, N, F_OUT), jnp.float32),
        compiler_params=pltpu.CompilerParams(
            dimension_semantics=("parallel",)),
    )(x1, x2, W, b2)
    return out[None]


# speedup vs baseline: 1.9886x; 1.7938x over previous
import jax
import jax.numpy as jnp
from jax.experimental import pallas as pl
from jax.experimental.pallas import tpu as pltpu

N = 256
F_IN = 128
F_OUT = 256
BI = 16


def _fused_body(x1_ref, x2_ref, w_ref, b_ref, out_ref):
    w = w_ref[...].astype(jnp.bfloat16)
    bb = b_ref[0]
    i = pl.program_id(0)
    e1b = jax.lax.dot_general(
        x1_ref[0, 0].astype(jnp.bfloat16), w, (((2,), (0,)), ((), ())),
        preferred_element_type=jnp.float32) + bb
    x2blk = x2_ref[0, 0, :, pl.ds(i * BI, BI), :].astype(jnp.bfloat16)
    e2b = jax.lax.dot_general(
        x2blk, w, (((2,), (0,)), ((), ())),
        preferred_element_type=jnp.float32) + bb
    out_ref[...] = jax.lax.dot_general(
        e1b.astype(jnp.bfloat16), e2b.astype(jnp.bfloat16),
        (((2,), (0,)), ((0,), (1,))),
        preferred_element_type=jnp.float32)


def kernel(x, W, b):
    b2 = b.reshape(1, F_OUT)
    out = pl.pallas_call(
        _fused_body,
        grid=(N // BI,),
        in_specs=[
            pl.BlockSpec((1, 1, BI, N, F_IN), lambda i: (0, 0, i, 0, 0)),
            pl.BlockSpec((1, 1, N, N, F_IN), lambda i: (0, 1, 0, 0, 0)),
            pl.BlockSpec((F_IN, F_OUT), lambda i: (0, 0)),
            pl.BlockSpec((1, F_OUT), lambda i: (0, 0)),
        ],
        out_specs=pl.BlockSpec((BI, N, F_OUT), lambda i: (i, 0, 0)),
        out_shape=jax.ShapeDtypeStruct((N, N, F_OUT), jnp.float32),
        compiler_params=pltpu.CompilerParams(
            dimension_semantics=("parallel",)),
    )(x, x, W, b2)
    return out[None]


# speedup vs baseline: 2.0717x; 1.0418x over previous
import jax
import jax.numpy as jnp
from jax.experimental import pallas as pl
from jax.experimental.pallas import tpu as pltpu

N = 256
F_IN = 128
F_OUT = 256
BI = 16


def _fused_body(x1_ref, x2_ref, w_ref, b_ref, out_ref):
    w = w_ref[...].astype(jnp.bfloat16)
    bb = b_ref[0]
    i = pl.program_id(0)
    e1b = jax.lax.dot_general(
        x1_ref[0, 0].astype(jnp.bfloat16), w, (((2,), (0,)), ((), ())),
        preferred_element_type=jnp.float32) + bb
    x2blk = x2_ref[0, 0].astype(jnp.bfloat16)
    e2b = jax.lax.dot_general(
        x2blk, w, (((2,), (0,)), ((), ())),
        preferred_element_type=jnp.float32) + bb
    out_ref[...] = jax.lax.dot_general(
        e1b.astype(jnp.bfloat16), e2b.astype(jnp.bfloat16),
        (((2,), (0,)), ((0,), (1,))),
        preferred_element_type=jnp.float32)


def kernel(x, W, b):
    b2 = b.reshape(1, F_OUT)
    out = pl.pallas_call(
        _fused_body,
        grid=(N // BI,),
        in_specs=[
            pl.BlockSpec((1, 1, BI, N, F_IN), lambda i: (0, 0, i, 0, 0)),
            pl.BlockSpec((1, 1, N, BI, F_IN), lambda i: (0, 1, 0, i, 0)),
            pl.BlockSpec((F_IN, F_OUT), lambda i: (0, 0)),
            pl.BlockSpec((1, F_OUT), lambda i: (0, 0)),
        ],
        out_specs=pl.BlockSpec((BI, N, F_OUT), lambda i: (i, 0, 0)),
        out_shape=jax.ShapeDtypeStruct((N, N, F_OUT), jnp.float32),
        compiler_params=pltpu.CompilerParams(
            dimension_semantics=("parallel",)),
    )(x, x, W, b2)
    return out[None]


# speedup vs baseline: 2.0830x; 1.0055x over previous
import jax
import jax.numpy as jnp
from jax.experimental import pallas as pl
from jax.experimental.pallas import tpu as pltpu

N = 256
F_IN = 128
F_OUT = 256
BI = 16


def _fused_body(x1_ref, x2_ref, w_ref, b_ref, out_ref):
    w = w_ref[...]
    wb = w.astype(jnp.bfloat16)
    bb = b_ref[0]
    x1b = x1_ref[0, 0].astype(jnp.bfloat16)
    x2b = x2_ref[0, 0].astype(jnp.bfloat16)
    s = jax.lax.dot_general(
        wb, x2b, (((1,), (0,)), ((), ())),
        preferred_element_type=jnp.float32)
    t = jax.lax.dot_general(
        s.astype(jnp.bfloat16), wb, (((2,), (0,)), ((), ())),
        preferred_element_type=jnp.float32)
    ws = jnp.sum(w, axis=1)
    tp = t + ws[:, None, None] * bb[None, None, :]
    v = jax.lax.dot_general(
        bb.astype(jnp.bfloat16), x2b, (((0,), (0,)), ((), ())),
        preferred_element_type=jnp.float32)
    u = jax.lax.dot_general(
        v.astype(jnp.bfloat16), wb, (((1,), (0,)), ((), ())),
        preferred_element_type=jnp.float32) + jnp.sum(bb) * bb[None, :]
    out_ref[...] = jax.lax.dot_general(
        x1b, tp.astype(jnp.bfloat16), (((2,), (0,)), ((0,), (1,))),
        preferred_element_type=jnp.float32) + u[:, None, :]


def kernel(x, W, b):
    b2 = b.reshape(1, F_OUT)
    out = pl.pallas_call(
        _fused_body,
        grid=(N // BI,),
        in_specs=[
            pl.BlockSpec((1, 1, BI, N, F_IN), lambda i: (0, 0, i, 0, 0)),
            pl.BlockSpec((1, 1, N, BI, F_IN), lambda i: (0, 1, 0, i, 0)),
            pl.BlockSpec((F_IN, F_OUT), lambda i: (0, 0)),
            pl.BlockSpec((1, F_OUT), lambda i: (0, 0)),
        ],
        out_specs=pl.BlockSpec((BI, N, F_OUT), lambda i: (i, 0, 0)),
        out_shape=jax.ShapeDtypeStruct((N, N, F_OUT), jnp.float32),
        compiler_params=pltpu.CompilerParams(
            dimension_semantics=("parallel",)),
    )(x, x, W, b2)
    return out[None]


# speedup vs baseline: 2.2462x; 1.0784x over previous
import jax
import jax.numpy as jnp
from jax.experimental import pallas as pl
from jax.experimental.pallas import tpu as pltpu

N = 256
F_IN = 128
F_OUT = 256
BI = 32


def _fused_body(x1_ref, x2_ref, w_ref, b_ref, out_ref):
    w = w_ref[...]
    wb = w.astype(jnp.bfloat16)
    bb = b_ref[0]
    x1b = x1_ref[0, 0].astype(jnp.bfloat16)
    x2b = x2_ref[0, 0].astype(jnp.bfloat16)
    s = jax.lax.dot_general(
        x2b, wb, (((0,), (1,)), ((), ())),
        preferred_element_type=jnp.float32)
    t = jax.lax.dot_general(
        s.astype(jnp.bfloat16), wb, (((1,), (0,)), ((), ())),
        preferred_element_type=jnp.float32)
    ws = jnp.sum(w, axis=1)
    tp = t + ws[None, :, None] * bb[None, None, :]
    v = jax.lax.dot_general(
        bb.astype(jnp.bfloat16), x2b, (((0,), (0,)), ((), ())),
        preferred_element_type=jnp.float32)
    u = jax.lax.dot_general(
        v.astype(jnp.bfloat16), wb, (((1,), (0,)), ((), ())),
        preferred_element_type=jnp.float32) + jnp.sum(bb) * bb[None, :]
    out_ref[...] = jax.lax.dot_general(
        x1b, tp.astype(jnp.bfloat16), (((2,), (1,)), ((0,), (0,))),
        preferred_element_type=jnp.float32) + u[:, None, :]


def kernel(x, W, b):
    b2 = b.reshape(1, F_OUT)
    out = pl.pallas_call(
        _fused_body,
        grid=(N // BI,),
        in_specs=[
            pl.BlockSpec((1, 1, BI, N, F_IN), lambda i: (0, 0, i, 0, 0)),
            pl.BlockSpec((1, 1, N, BI, F_IN), lambda i: (0, 1, 0, i, 0)),
            pl.BlockSpec((F_IN, F_OUT), lambda i: (0, 0)),
            pl.BlockSpec((1, F_OUT), lambda i: (0, 0)),
        ],
        out_specs=pl.BlockSpec((BI, N, F_OUT), lambda i: (i, 0, 0)),
        out_shape=jax.ShapeDtypeStruct((N, N, F_OUT), jnp.float32),
        compiler_params=pltpu.CompilerParams(
            dimension_semantics=("parallel",)),
    )(x, x, W, b2)
    return out[None]
